# Initial kernel scaffold; baseline (speedup 1.0000x reference)
import functools

import jax
import jax.numpy as jnp
from jax import lax
from jax.experimental import pallas as pl
from jax.experimental.pallas import tpu as pltpu
from jax.experimental.pallas import tpu_sc as plsc

N_NODES = 10000
D_FEAT = 128
N_EDGES = 320000

BLK = 128
NBLK = N_EDGES // BLK
NC = 2
NS = 16
NW = NC * NS
ROWS_PER_TILE = N_NODES // NS


def _elu_body(x_ref, w_ref, o_ref):
    z = x_ref[...] * w_ref[...]
    o_ref[...] = jnp.where(z > 0, z, jnp.expm1(z))


def _elu(x, w):
    return pl.pallas_call(
        _elu_body,
        grid=(10,),
        in_specs=[
            pl.BlockSpec((1000, D_FEAT), lambda i: (i, 0)),
            pl.BlockSpec((1, D_FEAT), lambda i: (0, 0)),
        ],
        out_specs=pl.BlockSpec((1000, D_FEAT), lambda i: (i, 0)),
        out_shape=jax.ShapeDtypeStruct((N_NODES, D_FEAT), jnp.float32),
    )(x, w)


def _combine_body(p_ref, o_ref):
    o_ref[...] = p_ref[0] + p_ref[1]


def _combine(p):
    return pl.pallas_call(
        _combine_body,
        grid=(10,),
        in_specs=[pl.BlockSpec((2, 1000, D_FEAT), lambda i: (0, i, 0))],
        out_specs=pl.BlockSpec((1000, D_FEAT), lambda i: (i, 0)),
        out_shape=jax.ShapeDtypeStruct((N_NODES, D_FEAT), jnp.float32),
    )(p)


def _sc_scatter(emb, src, dst, zrows):
    mesh = plsc.VectorSubcoreMesh(core_axis_name="c", subcore_axis_name="s")

    @functools.partial(
        pl.kernel,
        out_type=jax.ShapeDtypeStruct((NC, N_NODES, D_FEAT), jnp.float32),
        mesh=mesh,
        scratch_types=[
            pltpu.VMEM_SHARED((N_NODES, D_FEAT), jnp.float32),
            pltpu.VMEM((BLK,), jnp.int32),
            pltpu.VMEM((BLK,), jnp.int32),
            pltpu.VMEM((BLK, D_FEAT), jnp.float32),
            pltpu.VMEM((BLK, D_FEAT), jnp.float32),
            pltpu.SemaphoreType.DMA,
            pltpu.SemaphoreType.DMA,
        ],
    )
    def k(emb_hbm, src_hbm, dst_hbm, zrows_hbm, out_hbm,
          acc, sidx, didx, srows, drows, sem1, sem2):
        cid = lax.axis_index("c")
        sid = lax.axis_index("s")
        wid = sid * NC + cid

        base = sid * ROWS_PER_TILE
        pltpu.sync_copy(zrows_hbm, acc.at[pl.ds(base, ROWS_PER_TILE)])
        plsc.subcore_barrier()

        nblk = NBLK // NW + jnp.where(wid < NBLK - (NBLK // NW) * NW, 1, 0)

        def body(j, _):
            eoff = (wid + j * NW) * BLK
            pltpu.sync_copy(src_hbm.at[pl.ds(eoff, BLK)], sidx)
            pltpu.sync_copy(dst_hbm.at[pl.ds(eoff, BLK)], didx)
            cp1 = pltpu.async_copy(emb_hbm.at[sidx], srows, sem1)
            cp2 = pltpu.async_copy(emb_hbm.at[didx], drows, sem2)
            cp1.wait()
            cp2.wait()
            pltpu.sync_copy(srows, acc.at[didx], add=True)
            pltpu.sync_copy(drows, acc.at[sidx], add=True)
            return ()

        lax.fori_loop(0, nblk, body, ())
        plsc.subcore_barrier()

        pltpu.sync_copy(acc.at[pl.ds(base, ROWS_PER_TILE)],
                        out_hbm.at[cid, pl.ds(base, ROWS_PER_TILE)])

    return k(emb, src, dst, zrows)


def kernel(graph_embedding, edge_index, weight):
    emb = _elu(graph_embedding, weight)
    ei = edge_index.astype(jnp.int32)
    zrows = jnp.zeros((ROWS_PER_TILE, D_FEAT), jnp.float32)
    partial = _sc_scatter(emb, ei[0], ei[1], zrows)
    return _combine(partial)


# speedup vs baseline: 6.6759x; 6.6759x over previous
import functools

import jax
import jax.numpy as jnp
from jax import lax
from jax.experimental import pallas as pl
from jax.experimental.pallas import tpu as pltpu
from jax.experimental.pallas import tpu_sc as plsc

N_NODES = 10000
D_FEAT = 128
N_EDGES = 320000
N_PAD = 10240

BLK = 128
NBLK = N_EDGES // BLK
NC = 2
NS = 16
NW = NC * NS
ROWS_PER_TILE = N_PAD // NS


def _elu_body(x_ref, w_ref, o_ref):
    z = x_ref[...] * w_ref[...]
    o_ref[...] = jnp.where(z > 0, z, jnp.exp(z) - 1.0)


def _elu(x, w):
    return pl.pallas_call(
        _elu_body,
        grid=(10,),
        in_specs=[
            pl.BlockSpec((1000, D_FEAT), lambda i: (i, 0)),
            pl.BlockSpec((1, D_FEAT), lambda i: (0, 0)),
        ],
        out_specs=pl.BlockSpec((1000, D_FEAT), lambda i: (i, 0)),
        out_shape=jax.ShapeDtypeStruct((N_NODES, D_FEAT), jnp.float32),
    )(x, w)


def _combine_body(p_ref, o_ref):
    o_ref[...] = p_ref[0] + p_ref[1]


def _combine(p):
    return pl.pallas_call(
        _combine_body,
        grid=(10,),
        in_specs=[pl.BlockSpec((2, 1000, D_FEAT), lambda i: (0, i, 0))],
        out_specs=pl.BlockSpec((1000, D_FEAT), lambda i: (i, 0)),
        out_shape=jax.ShapeDtypeStruct((N_NODES, D_FEAT), jnp.float32),
    )(p)


def _sc_scatter(emb, src, dst, zrows):
    mesh = plsc.VectorSubcoreMesh(core_axis_name="c", subcore_axis_name="s")

    @functools.partial(
        pl.kernel,
        out_type=jax.ShapeDtypeStruct((NC, N_PAD, D_FEAT), jnp.float32),
        mesh=mesh,
        scratch_types=[
            pltpu.VMEM_SHARED((N_PAD, D_FEAT), jnp.float32),
            pltpu.VMEM((BLK,), jnp.int32),
            pltpu.VMEM((BLK,), jnp.int32),
            pltpu.VMEM((BLK, D_FEAT), jnp.float32),
            pltpu.VMEM((BLK, D_FEAT), jnp.float32),
            pltpu.SemaphoreType.DMA,
            pltpu.SemaphoreType.DMA,
        ],
    )
    def k(emb_hbm, src_hbm, dst_hbm, zrows_hbm, out_hbm,
          acc, sidx, didx, srows, drows, sem1, sem2):
        cid = lax.axis_index("c")
        sid = lax.axis_index("s")
        wid = sid * NC + cid

        base = sid * ROWS_PER_TILE
        pltpu.sync_copy(zrows_hbm, acc.at[pl.ds(base, ROWS_PER_TILE)])
        plsc.subcore_barrier()

        nblk = NBLK // NW + jnp.where(wid < NBLK - (NBLK // NW) * NW, 1, 0)

        def body(j, _):
            eoff = (wid + j * NW) * BLK
            pltpu.sync_copy(src_hbm.at[pl.ds(eoff, BLK)], sidx)
            pltpu.sync_copy(dst_hbm.at[pl.ds(eoff, BLK)], didx)
            cp1 = pltpu.async_copy(emb_hbm.at[sidx], srows, sem1)
            cp2 = pltpu.async_copy(emb_hbm.at[didx], drows, sem2)
            cp1.wait()
            cp2.wait()
            pltpu.sync_copy(srows, acc.at[didx], add=True)
            pltpu.sync_copy(drows, acc.at[sidx], add=True)
            return ()

        lax.fori_loop(0, nblk, body, ())
        plsc.subcore_barrier()

        pltpu.sync_copy(acc.at[pl.ds(base, ROWS_PER_TILE)],
                        out_hbm.at[cid, pl.ds(base, ROWS_PER_TILE)])

    return k(emb, src, dst, zrows)


def kernel(graph_embedding, edge_index, weight):
    emb = _elu(graph_embedding, weight)
    ei = edge_index.astype(jnp.int32)
    zrows = jnp.zeros((ROWS_PER_TILE, D_FEAT), jnp.float32)
    partial = _sc_scatter(emb, ei[0], ei[1], zrows)
    return _combine(partial)


# speedup vs baseline: 9.2004x; 1.3782x over previous
import functools

import jax
import jax.numpy as jnp
from jax import lax
from jax.experimental import pallas as pl
from jax.experimental.pallas import tpu as pltpu
from jax.experimental.pallas import tpu_sc as plsc

N_NODES = 10000
D_FEAT = 128
N_EDGES = 320000
N_PAD = 10240

BLK = 128
NC = 2
NS = 16
NW = NC * NS
BPW = 80
NBLK = NW * BPW
E_PAD = NBLK * BLK
ROWS_PER_TILE = N_PAD // NS


def _elu_body(x_ref, w_ref, o_ref):
    z = x_ref[...] * w_ref[...]
    o_ref[...] = jnp.where(z > 0, z, jnp.exp(z) - 1.0)


def _elu(x, w):
    return pl.pallas_call(
        _elu_body,
        grid=(10,),
        in_specs=[
            pl.BlockSpec((1000, D_FEAT), lambda i: (i, 0)),
            pl.BlockSpec((1, D_FEAT), lambda i: (0, 0)),
        ],
        out_specs=pl.BlockSpec((1000, D_FEAT), lambda i: (i, 0)),
        out_shape=jax.ShapeDtypeStruct((N_NODES, D_FEAT), jnp.float32),
    )(x, w)


def _combine_body(p_ref, o_ref):
    o_ref[...] = p_ref[0] + p_ref[1]


def _combine(p):
    return pl.pallas_call(
        _combine_body,
        grid=(10,),
        in_specs=[pl.BlockSpec((2, 1000, D_FEAT), lambda i: (0, i, 0))],
        out_specs=pl.BlockSpec((1000, D_FEAT), lambda i: (i, 0)),
        out_shape=jax.ShapeDtypeStruct((N_NODES, D_FEAT), jnp.float32),
    )(p)


def _sc_scatter(emb, src2d, dst2d, zrows):
    mesh = plsc.VectorSubcoreMesh(core_axis_name="c", subcore_axis_name="s")

    @functools.partial(
        pl.kernel,
        out_type=jax.ShapeDtypeStruct((NC, N_PAD, D_FEAT), jnp.float32),
        mesh=mesh,
        scratch_types=[
            pltpu.VMEM_SHARED((N_PAD, D_FEAT), jnp.float32),
            pltpu.VMEM((BPW // 2, BLK), jnp.int32),
            pltpu.VMEM((BPW // 2, BLK), jnp.int32),
            pltpu.VMEM((2, BLK, D_FEAT), jnp.float32),
            pltpu.SemaphoreType.DMA,
            pltpu.SemaphoreType.DMA,
            pltpu.SemaphoreType.DMA,
            pltpu.SemaphoreType.DMA,
        ],
    )
    def k(emb_hbm, src_hbm, dst_hbm, zrows_hbm, out_hbm,
          acc, sidx, didx, rows, gsem0, gsem1, ssem0, ssem1):
        cid = lax.axis_index("c")
        sid = lax.axis_index("s")
        wid = sid * NC + cid

        base = sid * ROWS_PER_TILE
        pltpu.sync_copy(zrows_hbm, acc.at[pl.ds(base, ROWS_PER_TILE)])
        plsc.subcore_barrier()

        gsems = (gsem0, gsem1)
        ssems = (ssem0, ssem1)
        half = BPW // 2

        def item(gidx, scat_idx, j, b):
            pltpu.make_async_copy(rows.at[b], acc.at[scat_idx.at[j - 1]],
                                  ssems[b]).wait()
            pltpu.async_copy(emb_hbm.at[gidx.at[j]], rows.at[b],
                             gsems[b]).wait()
            pltpu.async_copy(rows.at[b], acc.at[scat_idx.at[j]],
                             ssems[b], add=True)

        def item_first(gidx, scat_idx, b):
            pltpu.async_copy(emb_hbm.at[gidx.at[0]], rows.at[b],
                             gsems[b]).wait()
            pltpu.async_copy(rows.at[b], acc.at[scat_idx.at[0]],
                             ssems[b], add=True)

        for p in (0, 1):
            pltpu.sync_copy(src_hbm.at[pl.ds(wid * BPW + p * half, half)], sidx)
            pltpu.sync_copy(dst_hbm.at[pl.ds(wid * BPW + p * half, half)], didx)

            item_first(sidx, didx, 0)
            item_first(didx, sidx, 1)

            def body(j, _):
                item(sidx, didx, j, 0)
                item(didx, sidx, j, 1)
                return ()

            lax.fori_loop(1, half, body, ())
            pltpu.make_async_copy(rows.at[0], acc.at[didx.at[half - 1]],
                                  ssems[0]).wait()
            pltpu.make_async_copy(rows.at[1], acc.at[sidx.at[half - 1]],
                                  ssems[1]).wait()
        plsc.subcore_barrier()

        pltpu.sync_copy(acc.at[pl.ds(base, ROWS_PER_TILE)],
                        out_hbm.at[cid, pl.ds(base, ROWS_PER_TILE)])

    return k(emb, src2d, dst2d, zrows)


def kernel(graph_embedding, edge_index, weight):
    emb = _elu(graph_embedding, weight)
    ei = edge_index.astype(jnp.int32)
    n_extra = E_PAD - N_EDGES
    pad_src = jnp.arange(n_extra, dtype=jnp.int32) % N_NODES
    pad_dst = N_NODES + (jnp.arange(n_extra, dtype=jnp.int32) % (N_PAD - N_NODES))
    src2d = jnp.concatenate([ei[0], pad_src]).reshape(NBLK, BLK)
    dst2d = jnp.concatenate([ei[1], pad_dst]).reshape(NBLK, BLK)
    zrows = jnp.zeros((ROWS_PER_TILE, D_FEAT), jnp.float32)
    partial = _sc_scatter(emb, src2d, dst2d, zrows)
    return _combine(partial)


# speedup vs baseline: 10.4433x; 1.1351x over previous
import functools

import jax
import jax.numpy as jnp
from jax import lax
from jax.experimental import pallas as pl
from jax.experimental.pallas import tpu as pltpu
from jax.experimental.pallas import tpu_sc as plsc

N_NODES = 10000
D_FEAT = 128
HALF_D = D_FEAT // 2
N_EDGES = 320000
N_PAD = 10240

BLK = 128
NC = 2
NS = 16
TPB = 160
NBLK = NS * TPB
E_PAD = NBLK * BLK
PHB = TPB // 2
ROWS_PER_TILE = N_PAD // NS


def _elu_body(x_ref, w_ref, o_ref):
    z = x_ref[...] * w_ref[...]
    e = jnp.where(z > 0, z, jnp.exp(z) - 1.0)
    o_ref[0] = e[:, :HALF_D]
    o_ref[1] = e[:, HALF_D:]


def _elu(x, w):
    return pl.pallas_call(
        _elu_body,
        grid=(10,),
        in_specs=[
            pl.BlockSpec((1000, D_FEAT), lambda i: (i, 0)),
            pl.BlockSpec((1, D_FEAT), lambda i: (0, 0)),
        ],
        out_specs=pl.BlockSpec((2, 1000, HALF_D), lambda i: (0, i, 0)),
        out_shape=jax.ShapeDtypeStruct((2, N_NODES, HALF_D), jnp.float32),
    )(x, w)


def _assemble_body(p_ref, o_ref):
    o_ref[:, :HALF_D] = p_ref[0]
    o_ref[:, HALF_D:] = p_ref[1]


def _assemble(p):
    return pl.pallas_call(
        _assemble_body,
        grid=(10,),
        in_specs=[pl.BlockSpec((2, 1000, HALF_D), lambda i: (0, i, 0))],
        out_specs=pl.BlockSpec((1000, D_FEAT), lambda i: (i, 0)),
        out_shape=jax.ShapeDtypeStruct((N_NODES, D_FEAT), jnp.float32),
    )(p)


def _sc_scatter(emb2, src2d, dst2d, zrows):
    mesh = plsc.VectorSubcoreMesh(core_axis_name="c", subcore_axis_name="s")

    @functools.partial(
        pl.kernel,
        out_type=jax.ShapeDtypeStruct((NC, N_PAD, HALF_D), jnp.float32),
        mesh=mesh,
        compiler_params=pltpu.CompilerParams(use_tc_tiling_on_sc=False),
        scratch_types=[
            pltpu.VMEM_SHARED((N_PAD, HALF_D), jnp.float32),
            pltpu.VMEM((PHB, BLK), jnp.int32),
            pltpu.VMEM((PHB, BLK), jnp.int32),
            pltpu.VMEM((4, BLK, HALF_D), jnp.float32),
            pltpu.SemaphoreType.DMA,
            pltpu.SemaphoreType.DMA,
            pltpu.SemaphoreType.DMA,
            pltpu.SemaphoreType.DMA,
            pltpu.SemaphoreType.DMA,
            pltpu.SemaphoreType.DMA,
            pltpu.SemaphoreType.DMA,
            pltpu.SemaphoreType.DMA,
        ],
    )
    def k(emb_hbm, src_hbm, dst_hbm, zrows_hbm, out_hbm,
          acc, sidx, didx, rows,
          gsem0, gsem1, gsem2, gsem3, ssem0, ssem1, ssem2, ssem3):
        cid = lax.axis_index("c")
        sid = lax.axis_index("s")
        embc = emb_hbm.at[cid]
        gsems = (gsem0, gsem1, gsem2, gsem3)
        ssems = (ssem0, ssem1, ssem2, ssem3)

        base = sid * ROWS_PER_TILE
        pltpu.sync_copy(zrows_hbm, acc.at[pl.ds(base, ROWS_PER_TILE)])
        plsc.subcore_barrier()

        def g_start(gidx, j, b):
            pltpu.async_copy(embc.at[gidx.at[j]], rows.at[b], gsems[b])

        def g_wait(gidx, j, b):
            pltpu.make_async_copy(embc.at[gidx.at[j]], rows.at[b],
                                  gsems[b]).wait()

        def s_start(scat, j, b):
            pltpu.async_copy(rows.at[b], acc.at[scat.at[j]], ssems[b],
                             add=True)

        def s_wait(scat, j, b):
            pltpu.make_async_copy(rows.at[b], acc.at[scat.at[j]],
                                  ssems[b]).wait()

        for p in (0, 1):
            pltpu.sync_copy(src_hbm.at[pl.ds(sid * TPB + p * PHB, PHB)], sidx)
            pltpu.sync_copy(dst_hbm.at[pl.ds(sid * TPB + p * PHB, PHB)], didx)

            g_start(sidx, 0, 0)
            g_start(didx, 0, 1)
            g_start(sidx, 1, 2)
            g_wait(sidx, 0, 0)
            s_start(didx, 0, 0)
            g_start(didx, 1, 3)
            g_wait(didx, 0, 1)
            s_start(sidx, 0, 1)

            def body(jj, _):
                j0 = 2 * jj
                j1 = j0 + 1
                s_wait(didx, j0 - 2, 0)
                g_start(sidx, j0, 0)
                g_wait(sidx, j0 - 1, 2)
                s_start(didx, j0 - 1, 2)

                s_wait(sidx, j0 - 2, 1)
                g_start(didx, j0, 1)
                g_wait(didx, j0 - 1, 3)
                s_start(sidx, j0 - 1, 3)

                s_wait(didx, j1 - 2, 2)
                g_start(sidx, j1, 2)
                g_wait(sidx, j0, 0)
                s_start(didx, j0, 0)

                s_wait(sidx, j1 - 2, 3)
                g_start(didx, j1, 3)
                g_wait(didx, j0, 1)
                s_start(sidx, j0, 1)
                return ()

            lax.fori_loop(1, PHB // 2, body, ())

            last = PHB - 1
            g_wait(sidx, last, 2)
            s_start(didx, last, 2)
            g_wait(didx, last, 3)
            s_start(sidx, last, 3)
            s_wait(didx, last - 1, 0)
            s_wait(sidx, last - 1, 1)
            s_wait(didx, last, 2)
            s_wait(sidx, last, 3)

        plsc.subcore_barrier()

        pltpu.sync_copy(acc.at[pl.ds(base, ROWS_PER_TILE)],
                        out_hbm.at[cid, pl.ds(base, ROWS_PER_TILE)])

    return k(emb2, src2d, dst2d, zrows)


def kernel(graph_embedding, edge_index, weight):
    emb2 = _elu(graph_embedding, weight)
    ei = edge_index.astype(jnp.int32)
    n_extra = E_PAD - N_EDGES
    pad_src = jnp.arange(n_extra, dtype=jnp.int32) % N_NODES
    pad_dst = N_NODES + (jnp.arange(n_extra, dtype=jnp.int32) % (N_PAD - N_NODES))
    src2d = jnp.concatenate([ei[0], pad_src]).reshape(NBLK, BLK)
    dst2d = jnp.concatenate([ei[1], pad_dst]).reshape(NBLK, BLK)
    zrows = jnp.zeros((ROWS_PER_TILE, HALF_D), jnp.float32)
    partial = _sc_scatter(emb2, src2d, dst2d, zrows)
    return _assemble(partial)


# speedup vs baseline: 11.1873x; 1.0712x over previous
import functools

import jax
import jax.numpy as jnp
from jax import lax
from jax.experimental import pallas as pl
from jax.experimental.pallas import tpu as pltpu
from jax.experimental.pallas import tpu_sc as plsc

N_NODES = 10000
D_FEAT = 128
N_EDGES = 320000
N_ACC = 10016

BLK = 128
NC = 2
NS = 16
NW = NC * NS
BPW = 80
NBLK = NW * BPW
E_PAD = NBLK * BLK
ROWS_MAIN = 632
ROWS_LAST = N_ACC - 15 * ROWS_MAIN


def _elu_body(x_ref, w_ref, o_ref):
    z = x_ref[...] * w_ref[...]
    o_ref[...] = jnp.where(z > 0, z, jnp.exp(z) - 1.0)


def _elu(x, w):
    return pl.pallas_call(
        _elu_body,
        grid=(10,),
        in_specs=[
            pl.BlockSpec((1000, D_FEAT), lambda i: (i, 0)),
            pl.BlockSpec((1, D_FEAT), lambda i: (0, 0)),
        ],
        out_specs=pl.BlockSpec((1000, D_FEAT), lambda i: (i, 0)),
        out_shape=jax.ShapeDtypeStruct((N_NODES, D_FEAT), jnp.float32),
    )(x, w)


def _combine_body(p_ref, o_ref):
    o_ref[...] = p_ref[0] + p_ref[1]


def _combine(p):
    return pl.pallas_call(
        _combine_body,
        grid=(10,),
        in_specs=[pl.BlockSpec((2, 1000, D_FEAT), lambda i: (0, i, 0))],
        out_specs=pl.BlockSpec((1000, D_FEAT), lambda i: (i, 0)),
        out_shape=jax.ShapeDtypeStruct((N_NODES, D_FEAT), jnp.float32),
    )(p)


def _sc_scatter(emb, idx_il, zrows):
    mesh = plsc.VectorSubcoreMesh(core_axis_name="c", subcore_axis_name="s")

    @functools.partial(
        pl.kernel,
        out_type=jax.ShapeDtypeStruct((NC, N_ACC, D_FEAT), jnp.float32),
        mesh=mesh,
        compiler_params=pltpu.CompilerParams(use_tc_tiling_on_sc=False),
        scratch_types=[
            pltpu.VMEM_SHARED((N_ACC, D_FEAT), jnp.float32),
            pltpu.VMEM((6, 2, BLK), jnp.int32),
            pltpu.VMEM((3, BLK, D_FEAT), jnp.float32),
            pltpu.SemaphoreType.DMA,
            pltpu.SemaphoreType.DMA,
            pltpu.SemaphoreType.DMA,
            pltpu.SemaphoreType.DMA,
            pltpu.SemaphoreType.DMA,
            pltpu.SemaphoreType.DMA,
            pltpu.SemaphoreType.DMA,
            pltpu.SemaphoreType.DMA,
            pltpu.SemaphoreType.DMA,
            pltpu.SemaphoreType.DMA,
            pltpu.SemaphoreType.DMA,
            pltpu.SemaphoreType.DMA,
        ],
    )
    def k(emb_hbm, idx_hbm, zrows_hbm, out_hbm, acc, idxr, rows,
          gsem0, gsem1, gsem2, ssem0, ssem1, ssem2,
          isem0, isem1, isem2, isem3, isem4, isem5):
        cid = lax.axis_index("c")
        sid = lax.axis_index("s")
        wid = sid * NC + cid
        wb = wid * BPW
        gsems = (gsem0, gsem1, gsem2)
        ssems = (ssem0, ssem1, ssem2)
        isems = (isem0, isem1, isem2, isem3, isem4, isem5)

        @pl.when(sid < NS - 1)
        def _():
            pltpu.sync_copy(zrows_hbm,
                            acc.at[pl.ds(sid * ROWS_MAIN, ROWS_MAIN)])

        @pl.when(sid == NS - 1)
        def _():
            pltpu.sync_copy(zrows_hbm.at[pl.ds(0, ROWS_LAST)],
                            acc.at[pl.ds(15 * ROWS_MAIN, ROWS_LAST)])

        plsc.subcore_barrier()

        def i_start(j, slot):
            pltpu.async_copy(idx_hbm.at[pl.ds(2 * (wb + j), 2)],
                             idxr.at[slot], isems[slot])

        def i_wait(j, slot):
            pltpu.make_async_copy(idx_hbm.at[pl.ds(2 * (wb + j), 2)],
                                  idxr.at[slot], isems[slot]).wait()

        def g_start(j, d, slot, b):
            pltpu.async_copy(emb_hbm.at[idxr.at[slot, d]], rows.at[b],
                             gsems[b])

        def g_wait(j, d, slot, b):
            pltpu.make_async_copy(emb_hbm.at[idxr.at[slot, d]], rows.at[b],
                                  gsems[b]).wait()

        def s_start(j, d, slot, b):
            pltpu.async_copy(rows.at[b], acc.at[idxr.at[slot, 1 - d]],
                             ssems[b], add=True)

        def s_wait(j, d, slot, b):
            pltpu.make_async_copy(rows.at[b], acc.at[idxr.at[slot, 1 - d]],
                                  ssems[b]).wait()

        for j in range(4):
            i_start(j, j)
        i_wait(0, 0)
        g_start(0, 0, 0, 0)
        g_start(0, 1, 0, 1)
        g_wait(0, 0, 0, 0)
        s_start(0, 0, 0, 0)
        i_wait(1, 1)
        g_start(1, 0, 1, 2)
        g_wait(0, 1, 0, 1)
        s_start(0, 1, 0, 1)
        s_wait(0, 0, 0, 0)
        g_start(1, 1, 1, 0)
        g_wait(1, 0, 1, 2)
        s_start(1, 0, 1, 2)

        def body(m, _):
            jb = 2 + 6 * m
            for t in range(6):
                j = jb + t
                slot = (2 + t) % 6
                slot1 = (1 + t) % 6
                slot2 = (0 + t) % 6
                for d in (0, 1):
                    b = (4 + 2 * t + d) % 3
                    b1 = (3 + 2 * t + d) % 3
                    if d == 0:
                        s_wait(j - 2, 1, slot2, b)
                        i_wait(j, slot)
                        @pl.when(j < BPW - 2)
                        def _():
                            i_start(j + 2, (slot + 2) % 6)
                        g_start(j, 0, slot, b)
                        g_wait(j - 1, 1, slot1, b1)
                        s_start(j - 1, 1, slot1, b1)
                    else:
                        s_wait(j - 1, 0, slot1, b)
                        g_start(j, 1, slot, b)
                        g_wait(j, 0, slot, b1)
                        s_start(j, 0, slot, b1)
            return ()

        lax.fori_loop(0, 13, body, ())

        g_wait(79, 1, 1, 0)
        s_start(79, 1, 1, 0)
        s_wait(78, 1, 0, 1)
        s_wait(79, 0, 1, 2)
        s_wait(79, 1, 1, 0)
        plsc.subcore_barrier()

        @pl.when(sid < NS - 1)
        def _():
            pltpu.sync_copy(acc.at[pl.ds(sid * ROWS_MAIN, ROWS_MAIN)],
                            out_hbm.at[cid, pl.ds(sid * ROWS_MAIN, ROWS_MAIN)])

        @pl.when(sid == NS - 1)
        def _():
            pltpu.sync_copy(acc.at[pl.ds(15 * ROWS_MAIN, ROWS_LAST)],
                            out_hbm.at[cid, pl.ds(15 * ROWS_MAIN, ROWS_LAST)])

    return k(emb, idx_il, zrows)


def kernel(graph_embedding, edge_index, weight):
    emb = _elu(graph_embedding, weight)
    ei = edge_index.astype(jnp.int32)
    n_extra = E_PAD - N_EDGES
    pad_src = jnp.arange(n_extra, dtype=jnp.int32) % N_NODES
    pad_dst = N_NODES + (jnp.arange(n_extra, dtype=jnp.int32) % (N_ACC - N_NODES))
    src2d = jnp.concatenate([ei[0], pad_src]).reshape(NBLK, BLK)
    dst2d = jnp.concatenate([ei[1], pad_dst]).reshape(NBLK, BLK)
    idx_il = jnp.stack([src2d, dst2d], axis=1).reshape(2 * NBLK, BLK)
    zrows = jnp.zeros((ROWS_MAIN, D_FEAT), jnp.float32)
    partial = _sc_scatter(emb, idx_il, zrows)
    return _combine(partial)


# speedup vs baseline: 11.7369x; 1.0491x over previous
import functools

import jax
import jax.numpy as jnp
from jax import lax
from jax.experimental import pallas as pl
from jax.experimental.pallas import tpu as pltpu
from jax.experimental.pallas import tpu_sc as plsc

N_NODES = 10000
D_FEAT = 128
N_EDGES = 320000
N_ACC = 10016

BLK = 128
NC = 2
NS = 16
NW = NC * NS
BPW = 80
NBLK = NW * BPW
E_PAD = NBLK * BLK
ROWS_MAIN = 632
ROWS_LAST = N_ACC - 15 * ROWS_MAIN


def _prep_body(x_ref, w_ref, ei_ref, emb_ref, idx_ref):
    z = x_ref[...] * w_ref[...]
    emb_ref[...] = jnp.where(z > 0, z, jnp.exp(z) - 1.0)
    i = pl.program_id(0)
    rblk = i * 256 + lax.broadcasted_iota(jnp.int32, (256, BLK), 0)
    gid = rblk * BLK + lax.broadcasted_iota(jnp.int32, (256, BLK), 1)
    real = rblk < N_EDGES // BLK
    idx_ref[:, 0, :] = jnp.where(real, ei_ref[0], gid % N_NODES)
    idx_ref[:, 1, :] = jnp.where(real, ei_ref[1],
                                 N_NODES + gid % (N_ACC - N_NODES))


def _prep(x, w, ei3):
    return pl.pallas_call(
        _prep_body,
        grid=(10,),
        in_specs=[
            pl.BlockSpec((1000, D_FEAT), lambda i: (i, 0)),
            pl.BlockSpec((1, D_FEAT), lambda i: (0, 0)),
            pl.BlockSpec((2, 256, BLK), lambda i: (0, i, 0)),
        ],
        out_specs=[
            pl.BlockSpec((1000, D_FEAT), lambda i: (i, 0)),
            pl.BlockSpec((256, 2, BLK), lambda i: (i, 0, 0)),
        ],
        out_shape=[
            jax.ShapeDtypeStruct((N_NODES, D_FEAT), jnp.float32),
            jax.ShapeDtypeStruct((NBLK, 2, BLK), jnp.int32),
        ],
    )(x, w, ei3)


def _combine_body(p_ref, o_ref):
    o_ref[...] = p_ref[0] + p_ref[1]


def _combine(p):
    return pl.pallas_call(
        _combine_body,
        grid=(10,),
        in_specs=[pl.BlockSpec((2, 1000, D_FEAT), lambda i: (0, i, 0))],
        out_specs=pl.BlockSpec((1000, D_FEAT), lambda i: (i, 0)),
        out_shape=jax.ShapeDtypeStruct((N_NODES, D_FEAT), jnp.float32),
    )(p)


def _sc_scatter(emb, idx_il, zrows):
    mesh = plsc.VectorSubcoreMesh(core_axis_name="c", subcore_axis_name="s")

    @functools.partial(
        pl.kernel,
        out_type=jax.ShapeDtypeStruct((NC, N_ACC, D_FEAT), jnp.float32),
        mesh=mesh,
        compiler_params=pltpu.CompilerParams(use_tc_tiling_on_sc=False),
        scratch_types=[
            pltpu.VMEM_SHARED((N_ACC, D_FEAT), jnp.float32),
            pltpu.VMEM((6, 2, BLK), jnp.int32),
            pltpu.VMEM((3, BLK, D_FEAT), jnp.float32),
            pltpu.SemaphoreType.DMA,
            pltpu.SemaphoreType.DMA,
            pltpu.SemaphoreType.DMA,
            pltpu.SemaphoreType.DMA,
            pltpu.SemaphoreType.DMA,
            pltpu.SemaphoreType.DMA,
            pltpu.SemaphoreType.DMA,
            pltpu.SemaphoreType.DMA,
            pltpu.SemaphoreType.DMA,
            pltpu.SemaphoreType.DMA,
            pltpu.SemaphoreType.DMA,
            pltpu.SemaphoreType.DMA,
        ],
    )
    def k(emb_hbm, idx_hbm, zrows_hbm, out_hbm, acc, idxr, rows,
          gsem0, gsem1, gsem2, ssem0, ssem1, ssem2,
          isem0, isem1, isem2, isem3, isem4, isem5):
        cid = lax.axis_index("c")
        sid = lax.axis_index("s")
        wid = sid * NC + cid
        wb = wid * BPW
        gsems = (gsem0, gsem1, gsem2)
        ssems = (ssem0, ssem1, ssem2)
        isems = (isem0, isem1, isem2, isem3, isem4, isem5)

        @pl.when(sid < NS - 1)
        def _():
            pltpu.sync_copy(zrows_hbm,
                            acc.at[pl.ds(sid * ROWS_MAIN, ROWS_MAIN)])

        @pl.when(sid == NS - 1)
        def _():
            pltpu.sync_copy(zrows_hbm.at[pl.ds(0, ROWS_LAST)],
                            acc.at[pl.ds(15 * ROWS_MAIN, ROWS_LAST)])

        plsc.subcore_barrier()

        def i_start(j, slot):
            pltpu.async_copy(idx_hbm.at[wb + j], idxr.at[slot], isems[slot])

        def i_wait(j, slot):
            pltpu.make_async_copy(idx_hbm.at[wb + j], idxr.at[slot],
                                  isems[slot]).wait()

        def g_start(j, d, slot, b):
            pltpu.async_copy(emb_hbm.at[idxr.at[slot, d]], rows.at[b],
                             gsems[b])

        def g_wait(j, d, slot, b):
            pltpu.make_async_copy(emb_hbm.at[idxr.at[slot, d]], rows.at[b],
                                  gsems[b]).wait()

        def s_start(j, d, slot, b):
            pltpu.async_copy(rows.at[b], acc.at[idxr.at[slot, 1 - d]],
                             ssems[b], add=True)

        def s_wait(j, d, slot, b):
            pltpu.make_async_copy(rows.at[b], acc.at[idxr.at[slot, 1 - d]],
                                  ssems[b]).wait()

        for j in range(4):
            i_start(j, j)
        i_wait(0, 0)
        g_start(0, 0, 0, 0)
        g_start(0, 1, 0, 1)
        g_wait(0, 0, 0, 0)
        s_start(0, 0, 0, 0)
        i_wait(1, 1)
        g_start(1, 0, 1, 2)
        g_wait(0, 1, 0, 1)
        s_start(0, 1, 0, 1)
        s_wait(0, 0, 0, 0)
        g_start(1, 1, 1, 0)
        g_wait(1, 0, 1, 2)
        s_start(1, 0, 1, 2)

        def body(m, _):
            jb = 2 + 6 * m
            for t in range(6):
                j = jb + t
                slot = (2 + t) % 6
                slot1 = (1 + t) % 6
                slot2 = (0 + t) % 6
                for d in (0, 1):
                    b = (4 + 2 * t + d) % 3
                    b1 = (3 + 2 * t + d) % 3
                    if d == 0:
                        s_wait(j - 2, 1, slot2, b)
                        i_wait(j, slot)
                        @pl.when(j < BPW - 2)
                        def _():
                            i_start(j + 2, (slot + 2) % 6)
                        g_start(j, 0, slot, b)
                        g_wait(j - 1, 1, slot1, b1)
                        s_start(j - 1, 1, slot1, b1)
                    else:
                        s_wait(j - 1, 0, slot1, b)
                        g_start(j, 1, slot, b)
                        g_wait(j, 0, slot, b1)
                        s_start(j, 0, slot, b1)
            return ()

        lax.fori_loop(0, 13, body, ())

        g_wait(79, 1, 1, 0)
        s_start(79, 1, 1, 0)
        s_wait(78, 1, 0, 1)
        s_wait(79, 0, 1, 2)
        s_wait(79, 1, 1, 0)
        plsc.subcore_barrier()

        @pl.when(sid < NS - 1)
        def _():
            pltpu.sync_copy(acc.at[pl.ds(sid * ROWS_MAIN, ROWS_MAIN)],
                            out_hbm.at[cid, pl.ds(sid * ROWS_MAIN, ROWS_MAIN)])

        @pl.when(sid == NS - 1)
        def _():
            pltpu.sync_copy(acc.at[pl.ds(15 * ROWS_MAIN, ROWS_LAST)],
                            out_hbm.at[cid, pl.ds(15 * ROWS_MAIN, ROWS_LAST)])

    return k(emb, idx_il, zrows)


def kernel(graph_embedding, edge_index, weight):
    ei3 = edge_index.astype(jnp.int32).reshape(2, N_EDGES // BLK, BLK)
    emb, idx_il = _prep(graph_embedding, weight, ei3)
    zrows = jnp.zeros((ROWS_MAIN, D_FEAT), jnp.float32)
    partial = _sc_scatter(emb, idx_il, zrows)
    return _combine(partial)
